# baseline (speedup 1.0000x reference)
import jax
import jax.numpy as jnp
from jax.experimental import pallas as pl
from jax.experimental.pallas import tpu as pltpu

N = 4096
D = 128
IN_DIM = 384
BM = 512
NB = N // BM
EPS = 1e-12


def _dott(xt, a):
    return jax.lax.dot_general(
        xt.astype(jnp.bfloat16), a.astype(jnp.bfloat16),
        (((1,), (1,)), ((), ())), preferred_element_type=jnp.float32,
    )


def _proj_kernel(uf, bf, wu, wb, u0t, b0t):
    u0t[...] = jax.lax.dot_general(
        wu[...].astype(jnp.bfloat16), uf[...].astype(jnp.bfloat16),
        (((0,), (1,)), ((), ())), preferred_element_type=jnp.float32,
    )
    b0t[...] = jax.lax.dot_general(
        wb[...].astype(jnp.bfloat16), bf[...].astype(jnp.bfloat16),
        (((0,), (1,)), ((), ())), preferred_element_type=jnp.float32,
    )


def _sweep_kernel(abu, auu, aub, abb, u0r, b0r,
                  u1o, b1o, suo, sbo, u2p, b2p, u1s, b1s):
    i = pl.program_id(0)
    t = pl.program_id(1)
    j = jax.lax.rem(i + 1 + t, NB)
    row = pl.ds(i * BM, BM)
    col = pl.ds(j * BM, BM)

    @pl.when(t == 0)
    def _():
        u1s[:, row] = jnp.zeros((D, BM), jnp.float32)
        b1s[:, row] = jnp.zeros((D, BM), jnp.float32)
        u2p[...] = jnp.zeros_like(u2p)
        b2p[...] = jnp.zeros_like(b2p)

    u0j = u0r[:, col]
    b0j = b0r[:, col]

    @pl.when(j < i)
    def _():
        ustk = jnp.concatenate([u0j, u1s[:, col]], axis=0)
        bstk = jnp.concatenate([b0j, b1s[:, col]], axis=0)
        rbu = _dott(bstk, abu[...])
        ruu = _dott(ustk, auu[...])
        rub = _dott(ustk, aub[...])
        rbb = _dott(bstk, abb[...])
        u1s[:, row] += rbu[:D, :] + ruu[:D, :]
        b1s[:, row] += rub[:D, :] + rbb[:D, :]
        u2p[...] += rbu[D:, :] + ruu[D:, :]
        b2p[...] += rub[D:, :] + rbb[D:, :]

    @pl.when(j >= i)
    def _():
        u1s[:, row] += _dott(b0j, abu[...]) + _dott(u0j, auu[...])
        b1s[:, row] += _dott(u0j, aub[...]) + _dott(b0j, abb[...])

    @pl.when(t == NB - 1)
    def _():
        u1i = u1s[:, row]
        b1i = b1s[:, row]
        u2p[...] += _dott(b1i, abu[...]) + _dott(u1i, auu[...])
        b2p[...] += _dott(u1i, aub[...]) + _dott(b1i, abb[...])
        u1o[...] = u1i
        b1o[...] = b1i
        suo[...] = u0r[:, row] + u1i
        sbo[...] = b0r[:, row] + b1i


def _upper_kernel(abu, auu, aub, abb, u1r, b1r, su, sb, u2p, b2p,
                  uh, bh, accu, accb):
    i = pl.program_id(0)
    t = pl.program_id(1)

    @pl.when(t == 0)
    def _():
        accu[...] = u2p[...]
        accb[...] = b2p[...]

    @pl.when(t > i)
    def _():
        col = pl.ds(t * BM, BM)
        u1j = u1r[:, col]
        b1j = b1r[:, col]
        accu[...] += _dott(b1j, abu[...]) + _dott(u1j, auu[...])
        accb[...] += _dott(u1j, aub[...]) + _dott(b1j, abb[...])

    @pl.when(t == NB - 1)
    def _():
        emb_u = (su[...] + accu[...]) * (1.0 / 3.0)
        emb_b = (sb[...] + accb[...]) * (1.0 / 3.0)
        nu = jnp.sqrt(jnp.sum(emb_u * emb_u, axis=0, keepdims=True))
        nb = jnp.sqrt(jnp.sum(emb_b * emb_b, axis=0, keepdims=True))
        uh[...] = jnp.transpose(emb_u / jnp.maximum(nu, EPS))
        bh[...] = jnp.transpose(emb_b / jnp.maximum(nb, EPS))


def _rot_spec():
    return pl.BlockSpec((BM, BM), lambda i, t: (i, jax.lax.rem(i + 1 + t, NB)))


def _upper_spec():
    return pl.BlockSpec(
        (BM, BM),
        lambda i, t: (i, jnp.minimum(jnp.maximum(t, i + 1), NB - 1)),
    )


def _resident_spec():
    return pl.BlockSpec((D, N), lambda i, t: (0, 0))


def _col_spec():
    return pl.BlockSpec((D, BM), lambda i, t: (0, i))


def kernel(user_feat, biz_feat, adj_ub, adj_bu, adj_uu, adj_bb, W_user, W_biz):
    embt = jax.ShapeDtypeStruct((D, N), jnp.float32)

    u0t, b0t = pl.pallas_call(
        _proj_kernel,
        grid=(NB,),
        in_specs=[
            pl.BlockSpec((BM, IN_DIM), lambda i: (i, 0)),
            pl.BlockSpec((BM, IN_DIM), lambda i: (i, 0)),
            pl.BlockSpec((IN_DIM, D), lambda i: (0, 0)),
            pl.BlockSpec((IN_DIM, D), lambda i: (0, 0)),
        ],
        out_specs=[
            pl.BlockSpec((D, BM), lambda i: (0, i)),
            pl.BlockSpec((D, BM), lambda i: (0, i)),
        ],
        out_shape=[embt, embt],
        compiler_params=pltpu.CompilerParams(
            dimension_semantics=("parallel",),
        ),
    )(user_feat, biz_feat, W_user, W_biz)

    u1t, b1t, sut, sbt, u2p, b2p = pl.pallas_call(
        _sweep_kernel,
        grid=(NB, NB),
        in_specs=[
            _rot_spec(), _rot_spec(), _rot_spec(), _rot_spec(),
            _resident_spec(), _resident_spec(),
        ],
        out_specs=[_col_spec()] * 6,
        out_shape=[embt] * 6,
        scratch_shapes=[
            pltpu.VMEM((D, N), jnp.float32),
            pltpu.VMEM((D, N), jnp.float32),
        ],
        compiler_params=pltpu.CompilerParams(
            dimension_semantics=("arbitrary", "arbitrary"),
        ),
    )(adj_bu, adj_uu, adj_ub, adj_bb, u0t, b0t)

    user_h, biz_h = pl.pallas_call(
        _upper_kernel,
        grid=(NB, NB),
        in_specs=[
            _upper_spec(), _upper_spec(), _upper_spec(), _upper_spec(),
            _resident_spec(), _resident_spec(),
            _col_spec(), _col_spec(), _col_spec(), _col_spec(),
        ],
        out_specs=[
            pl.BlockSpec((BM, D), lambda i, t: (i, 0)),
            pl.BlockSpec((BM, D), lambda i, t: (i, 0)),
        ],
        out_shape=[
            jax.ShapeDtypeStruct((N, D), jnp.float32),
            jax.ShapeDtypeStruct((N, D), jnp.float32),
        ],
        scratch_shapes=[
            pltpu.VMEM((D, BM), jnp.float32),
            pltpu.VMEM((D, BM), jnp.float32),
        ],
        compiler_params=pltpu.CompilerParams(
            dimension_semantics=("parallel", "arbitrary"),
        ),
    )(adj_bu, adj_uu, adj_ub, adj_bb, u1t, b1t, sut, sbt, u2p, b2p)

    return (user_h, biz_h)


# speedup vs baseline: 1.0754x; 1.0754x over previous
import jax
import jax.numpy as jnp
from jax.experimental import pallas as pl
from jax.experimental.pallas import tpu as pltpu

N = 4096
D = 128
IN_DIM = 384
BM = 512
NB = N // BM
S1 = NB * NB
SU = NB * (NB - 1) // 2 + 1
EPS = 1e-12


def _dott(xt, a):
    return jax.lax.dot_general(
        xt.astype(jnp.bfloat16), a.astype(jnp.bfloat16),
        (((1,), (1,)), ((), ())), preferred_element_type=jnp.float32,
    )


def _split(s):
    in_sweep = s < S1
    i1 = s // NB
    t = s - i1 * NB
    j1 = jax.lax.rem(i1 + 1 + t, NB)
    p = jnp.maximum(s - S1, 0)
    i2 = jnp.int32(0)
    for r in range(1, NB):
        i2 = i2 + (p >= (r * (2 * NB - 1 - r) // 2)).astype(jnp.int32)
    off = (i2 * (2 * NB - 1 - i2)) // 2
    j2r = i2 + 1 + (p - off)
    j2 = jnp.minimum(j2r, NB - 1)
    i = jnp.where(in_sweep, i1, i2)
    j = jnp.where(in_sweep, j1, j2)
    fin = jnp.logical_and(jnp.logical_not(in_sweep), j2r >= NB - 1)
    return in_sweep, i, j, t, fin


def _proj_kernel(uf, bf, wu, wb, u0t, b0t):
    u0t[...] = jax.lax.dot_general(
        wu[...].astype(jnp.bfloat16), uf[...].astype(jnp.bfloat16),
        (((0,), (1,)), ((), ())), preferred_element_type=jnp.float32,
    )
    b0t[...] = jax.lax.dot_general(
        wb[...].astype(jnp.bfloat16), bf[...].astype(jnp.bfloat16),
        (((0,), (1,)), ((), ())), preferred_element_type=jnp.float32,
    )


def _prop_kernel(abu, auu, aub, abb, u0r, b0r, uh, bh, u1s, b1s, u2s, b2s):
    s = pl.program_id(0)
    in_sweep, i, j, t, fin = _split(s)
    row = pl.ds(i * BM, BM)
    col = pl.ds(j * BM, BM)

    @pl.when(in_sweep)
    def _():
        @pl.when(t == 0)
        def _():
            z = jnp.zeros((D, BM), jnp.float32)
            u1s[:, row] = z
            b1s[:, row] = z
            u2s[:, row] = z
            b2s[:, row] = z

        u0j = u0r[:, col]
        b0j = b0r[:, col]

        @pl.when(j < i)
        def _():
            ustk = jnp.concatenate([u0j, u1s[:, col]], axis=0)
            bstk = jnp.concatenate([b0j, b1s[:, col]], axis=0)
            rbu = _dott(bstk, abu[...])
            ruu = _dott(ustk, auu[...])
            rub = _dott(ustk, aub[...])
            rbb = _dott(bstk, abb[...])
            u1s[:, row] += rbu[:D, :] + ruu[:D, :]
            b1s[:, row] += rub[:D, :] + rbb[:D, :]
            u2s[:, row] += rbu[D:, :] + ruu[D:, :]
            b2s[:, row] += rub[D:, :] + rbb[D:, :]

        @pl.when(j >= i)
        def _():
            u1s[:, row] += _dott(b0j, abu[...]) + _dott(u0j, auu[...])
            b1s[:, row] += _dott(u0j, aub[...]) + _dott(b0j, abb[...])

        @pl.when(t == NB - 1)
        def _():
            u1i = u1s[:, row]
            b1i = b1s[:, row]
            u2s[:, row] += _dott(b1i, abu[...]) + _dott(u1i, auu[...])
            b2s[:, row] += _dott(u1i, aub[...]) + _dott(b1i, abb[...])

    @pl.when(jnp.logical_not(in_sweep))
    def _():
        @pl.when(j > i)
        def _():
            u1j = u1s[:, col]
            b1j = b1s[:, col]
            u2s[:, row] += _dott(b1j, abu[...]) + _dott(u1j, auu[...])
            b2s[:, row] += _dott(u1j, aub[...]) + _dott(b1j, abb[...])

        @pl.when(fin)
        def _():
            emb_u = (u0r[:, row] + u1s[:, row] + u2s[:, row]) * (1.0 / 3.0)
            emb_b = (b0r[:, row] + b1s[:, row] + b2s[:, row]) * (1.0 / 3.0)
            nu = jnp.sqrt(jnp.sum(emb_u * emb_u, axis=0, keepdims=True))
            nb = jnp.sqrt(jnp.sum(emb_b * emb_b, axis=0, keepdims=True))
            uh[...] = jnp.transpose(emb_u / jnp.maximum(nu, EPS))
            bh[...] = jnp.transpose(emb_b / jnp.maximum(nb, EPS))


def _adj_map(s):
    _, i, j, _, _ = _split(s)
    return (i, j)


def _out_map(s):
    in_sweep, i, _, _, _ = _split(s)
    return (jnp.where(in_sweep, 0, i), 0)


def kernel(user_feat, biz_feat, adj_ub, adj_bu, adj_uu, adj_bb, W_user, W_biz):
    embt = jax.ShapeDtypeStruct((D, N), jnp.float32)

    u0t, b0t = pl.pallas_call(
        _proj_kernel,
        grid=(NB,),
        in_specs=[
            pl.BlockSpec((BM, IN_DIM), lambda i: (i, 0)),
            pl.BlockSpec((BM, IN_DIM), lambda i: (i, 0)),
            pl.BlockSpec((IN_DIM, D), lambda i: (0, 0)),
            pl.BlockSpec((IN_DIM, D), lambda i: (0, 0)),
        ],
        out_specs=[
            pl.BlockSpec((D, BM), lambda i: (0, i)),
            pl.BlockSpec((D, BM), lambda i: (0, i)),
        ],
        out_shape=[embt, embt],
        compiler_params=pltpu.CompilerParams(
            dimension_semantics=("parallel",),
        ),
    )(user_feat, biz_feat, W_user, W_biz)

    adj_spec = pl.BlockSpec((BM, BM), _adj_map)
    res_spec = pl.BlockSpec((D, N), lambda s: (0, 0))
    out_spec = pl.BlockSpec((BM, D), _out_map)

    user_h, biz_h = pl.pallas_call(
        _prop_kernel,
        grid=(S1 + SU,),
        in_specs=[adj_spec, adj_spec, adj_spec, adj_spec, res_spec, res_spec],
        out_specs=[out_spec, out_spec],
        out_shape=[
            jax.ShapeDtypeStruct((N, D), jnp.float32),
            jax.ShapeDtypeStruct((N, D), jnp.float32),
        ],
        scratch_shapes=[
            pltpu.VMEM((D, N), jnp.float32),
            pltpu.VMEM((D, N), jnp.float32),
            pltpu.VMEM((D, N), jnp.float32),
            pltpu.VMEM((D, N), jnp.float32),
        ],
        compiler_params=pltpu.CompilerParams(
            dimension_semantics=("arbitrary",),
        ),
    )(adj_bu, adj_uu, adj_ub, adj_bb, u0t, b0t)

    return (user_h, biz_h)


# speedup vs baseline: 1.3441x; 1.2498x over previous
import jax
import jax.numpy as jnp
from jax.experimental import pallas as pl
from jax.experimental.pallas import tpu as pltpu

N = 4096
D = 128
IN_DIM = 384
BM = 1024
NB = N // BM
S1 = NB * NB
SU = NB * (NB - 1) // 2 + 1
EPS = 1e-12


def _dott(xt, a):
    return jax.lax.dot_general(
        xt.astype(jnp.bfloat16), a.astype(jnp.bfloat16),
        (((1,), (1,)), ((), ())), preferred_element_type=jnp.float32,
    )


def _split(s):
    in_sweep = s < S1
    i1 = s // NB
    t = s - i1 * NB
    j1 = jax.lax.rem(i1 + 1 + t, NB)
    p = jnp.maximum(s - S1, 0)
    i2 = jnp.int32(0)
    for r in range(1, NB):
        i2 = i2 + (p >= (r * (2 * NB - 1 - r) // 2)).astype(jnp.int32)
    off = (i2 * (2 * NB - 1 - i2)) // 2
    j2r = i2 + 1 + (p - off)
    j2 = jnp.minimum(j2r, NB - 1)
    i = jnp.where(in_sweep, i1, i2)
    j = jnp.where(in_sweep, j1, j2)
    fin = jnp.logical_and(jnp.logical_not(in_sweep), j2r >= NB - 1)
    return in_sweep, i, j, t, fin


def _proj_kernel(uf, bf, wu, wb, u0t, b0t):
    u0t[...] = jax.lax.dot_general(
        wu[...].astype(jnp.bfloat16), uf[...].astype(jnp.bfloat16),
        (((0,), (1,)), ((), ())), preferred_element_type=jnp.float32,
    )
    b0t[...] = jax.lax.dot_general(
        wb[...].astype(jnp.bfloat16), bf[...].astype(jnp.bfloat16),
        (((0,), (1,)), ((), ())), preferred_element_type=jnp.float32,
    )


def _prop_kernel(abu, auu, aub, abb, u0r, b0r, uh, bh, u1s, b1s, u2s, b2s):
    s = pl.program_id(0)
    in_sweep, i, j, t, fin = _split(s)
    row = pl.ds(i * BM, BM)
    col = pl.ds(j * BM, BM)

    @pl.when(in_sweep)
    def _():
        @pl.when(t == 0)
        def _():
            z = jnp.zeros((D, BM), jnp.float32)
            u1s[:, row] = z
            b1s[:, row] = z
            u2s[:, row] = z
            b2s[:, row] = z

        u0j = u0r[:, col]
        b0j = b0r[:, col]

        @pl.when(j < i)
        def _():
            ustk = jnp.concatenate([u0j, u1s[:, col]], axis=0)
            bstk = jnp.concatenate([b0j, b1s[:, col]], axis=0)
            rbu = _dott(bstk, abu[...])
            ruu = _dott(ustk, auu[...])
            rub = _dott(ustk, aub[...])
            rbb = _dott(bstk, abb[...])
            u1s[:, row] += rbu[:D, :] + ruu[:D, :]
            b1s[:, row] += rub[:D, :] + rbb[:D, :]
            u2s[:, row] += rbu[D:, :] + ruu[D:, :]
            b2s[:, row] += rub[D:, :] + rbb[D:, :]

        @pl.when(j >= i)
        def _():
            u1s[:, row] += _dott(b0j, abu[...]) + _dott(u0j, auu[...])
            b1s[:, row] += _dott(u0j, aub[...]) + _dott(b0j, abb[...])

        @pl.when(t == NB - 1)
        def _():
            u1i = u1s[:, row]
            b1i = b1s[:, row]
            u2s[:, row] += _dott(b1i, abu[...]) + _dott(u1i, auu[...])
            b2s[:, row] += _dott(u1i, aub[...]) + _dott(b1i, abb[...])

    @pl.when(jnp.logical_not(in_sweep))
    def _():
        @pl.when(j > i)
        def _():
            u1j = u1s[:, col]
            b1j = b1s[:, col]
            u2s[:, row] += _dott(b1j, abu[...]) + _dott(u1j, auu[...])
            b2s[:, row] += _dott(u1j, aub[...]) + _dott(b1j, abb[...])

        @pl.when(fin)
        def _():
            emb_u = (u0r[:, row] + u1s[:, row] + u2s[:, row]) * (1.0 / 3.0)
            emb_b = (b0r[:, row] + b1s[:, row] + b2s[:, row]) * (1.0 / 3.0)
            nu = jnp.sqrt(jnp.sum(emb_u * emb_u, axis=0, keepdims=True))
            nb = jnp.sqrt(jnp.sum(emb_b * emb_b, axis=0, keepdims=True))
            uh[...] = jnp.transpose(emb_u / jnp.maximum(nu, EPS))
            bh[...] = jnp.transpose(emb_b / jnp.maximum(nb, EPS))


def _adj_map(s):
    _, i, j, _, _ = _split(s)
    return (i, j)


def _out_map(s):
    in_sweep, i, _, _, _ = _split(s)
    return (jnp.where(in_sweep, 0, i), 0)


def kernel(user_feat, biz_feat, adj_ub, adj_bu, adj_uu, adj_bb, W_user, W_biz):
    embt = jax.ShapeDtypeStruct((D, N), jnp.float32)

    u0t, b0t = pl.pallas_call(
        _proj_kernel,
        grid=(NB,),
        in_specs=[
            pl.BlockSpec((BM, IN_DIM), lambda i: (i, 0)),
            pl.BlockSpec((BM, IN_DIM), lambda i: (i, 0)),
            pl.BlockSpec((IN_DIM, D), lambda i: (0, 0)),
            pl.BlockSpec((IN_DIM, D), lambda i: (0, 0)),
        ],
        out_specs=[
            pl.BlockSpec((D, BM), lambda i: (0, i)),
            pl.BlockSpec((D, BM), lambda i: (0, i)),
        ],
        out_shape=[embt, embt],
        compiler_params=pltpu.CompilerParams(
            dimension_semantics=("parallel",),
        ),
    )(user_feat, biz_feat, W_user, W_biz)

    adj_spec = pl.BlockSpec((BM, BM), _adj_map)
    res_spec = pl.BlockSpec((D, N), lambda s: (0, 0))
    out_spec = pl.BlockSpec((BM, D), _out_map)

    user_h, biz_h = pl.pallas_call(
        _prop_kernel,
        grid=(S1 + SU,),
        in_specs=[adj_spec, adj_spec, adj_spec, adj_spec, res_spec, res_spec],
        out_specs=[out_spec, out_spec],
        out_shape=[
            jax.ShapeDtypeStruct((N, D), jnp.float32),
            jax.ShapeDtypeStruct((N, D), jnp.float32),
        ],
        scratch_shapes=[
            pltpu.VMEM((D, N), jnp.float32),
            pltpu.VMEM((D, N), jnp.float32),
            pltpu.VMEM((D, N), jnp.float32),
            pltpu.VMEM((D, N), jnp.float32),
        ],
        compiler_params=pltpu.CompilerParams(
            dimension_semantics=("arbitrary",),
        ),
    )(adj_bu, adj_uu, adj_ub, adj_bb, u0t, b0t)

    return (user_h, biz_h)


# speedup vs baseline: 1.3860x; 1.0312x over previous
import jax
import jax.numpy as jnp
from jax.experimental import pallas as pl
from jax.experimental.pallas import tpu as pltpu

N = 4096
D = 128
IN_DIM = 384
BM = 1024
NB = N // BM
S1 = NB * NB
SU = NB * (NB - 1) // 2 + 1
EPS = 1e-12


def _dott(xt, a):
    return jax.lax.dot_general(
        xt.astype(jnp.bfloat16), a.astype(jnp.bfloat16),
        (((1,), (1,)), ((), ())), preferred_element_type=jnp.float32,
    )


def _split(s):
    in_sweep = s < S1
    i1 = s // NB
    t = s - i1 * NB
    j1 = jax.lax.rem(i1 + 1 + t, NB)
    p = jnp.maximum(s - S1, 0)
    i2 = jnp.int32(0)
    for r in range(1, NB):
        i2 = i2 + (p >= (r * (2 * NB - 1 - r) // 2)).astype(jnp.int32)
    off = (i2 * (2 * NB - 1 - i2)) // 2
    j2r = i2 + 1 + (p - off)
    j2 = jnp.minimum(j2r, NB - 1)
    i = jnp.where(in_sweep, i1, i2)
    j = jnp.where(in_sweep, j1, j2)
    fin = jnp.logical_and(jnp.logical_not(in_sweep), j2r >= NB - 1)
    dummy = jnp.logical_and(jnp.logical_not(in_sweep), j2r > NB - 1)
    return in_sweep, i, j, t, fin, dummy


def _kernel(abu, auu, aub, abb, uf, bf, wu, wb, uh, bh,
            u0s, b0s, u1s, b1s, u2s, b2s):
    s = pl.program_id(0)
    in_sweep, i, j, t, fin, _ = _split(s)
    row = pl.ds(i * BM, BM)
    col = pl.ds(j * BM, BM)

    @pl.when(in_sweep)
    def _():
        @pl.when(i == 0)
        def _():
            u0s[:, col] = jax.lax.dot_general(
                wu[...].astype(jnp.bfloat16), uf[...].astype(jnp.bfloat16),
                (((0,), (1,)), ((), ())), preferred_element_type=jnp.float32,
            )
            b0s[:, col] = jax.lax.dot_general(
                wb[...].astype(jnp.bfloat16), bf[...].astype(jnp.bfloat16),
                (((0,), (1,)), ((), ())), preferred_element_type=jnp.float32,
            )

        @pl.when(t == 0)
        def _():
            z = jnp.zeros((D, BM), jnp.float32)
            u1s[:, row] = z
            b1s[:, row] = z
            u2s[:, row] = z
            b2s[:, row] = z

        u0j = u0s[:, col]
        b0j = b0s[:, col]

        @pl.when(j < i)
        def _():
            ustk = jnp.concatenate([u0j, u1s[:, col]], axis=0)
            bstk = jnp.concatenate([b0j, b1s[:, col]], axis=0)
            rbu = _dott(bstk, abu[...])
            ruu = _dott(ustk, auu[...])
            rub = _dott(ustk, aub[...])
            rbb = _dott(bstk, abb[...])
            u1s[:, row] += rbu[:D, :] + ruu[:D, :]
            b1s[:, row] += rub[:D, :] + rbb[:D, :]
            u2s[:, row] += rbu[D:, :] + ruu[D:, :]
            b2s[:, row] += rub[D:, :] + rbb[D:, :]

        @pl.when(j >= i)
        def _():
            u1s[:, row] += _dott(b0j, abu[...]) + _dott(u0j, auu[...])
            b1s[:, row] += _dott(u0j, aub[...]) + _dott(b0j, abb[...])

        @pl.when(t == NB - 1)
        def _():
            u1i = u1s[:, row]
            b1i = b1s[:, row]
            u2s[:, row] += _dott(b1i, abu[...]) + _dott(u1i, auu[...])
            b2s[:, row] += _dott(u1i, aub[...]) + _dott(b1i, abb[...])

    @pl.when(jnp.logical_not(in_sweep))
    def _():
        @pl.when(j > i)
        def _():
            u1j = u1s[:, col]
            b1j = b1s[:, col]
            u2s[:, row] += _dott(b1j, abu[...]) + _dott(u1j, auu[...])
            b2s[:, row] += _dott(u1j, aub[...]) + _dott(b1j, abb[...])

        @pl.when(fin)
        def _():
            emb_u = (u0s[:, row] + u1s[:, row] + u2s[:, row]) * (1.0 / 3.0)
            emb_b = (b0s[:, row] + b1s[:, row] + b2s[:, row]) * (1.0 / 3.0)
            nu = jnp.sqrt(jnp.sum(emb_u * emb_u, axis=0, keepdims=True))
            nb = jnp.sqrt(jnp.sum(emb_b * emb_b, axis=0, keepdims=True))
            uh[...] = jnp.transpose(emb_u / jnp.maximum(nu, EPS))
            bh[...] = jnp.transpose(emb_b / jnp.maximum(nb, EPS))


def _adj_map(s):
    _, i, j, _, _, dummy = _split(s)
    return (jnp.where(dummy, i - 1, i), j)


def _feat_map(s):
    return (jnp.where(s < NB, jax.lax.rem(s + 1, NB), 0), 0)


def _out_map(s):
    in_sweep, i, _, _, _, _ = _split(s)
    return (jnp.where(in_sweep, 0, i), 0)


def kernel(user_feat, biz_feat, adj_ub, adj_bu, adj_uu, adj_bb, W_user, W_biz):
    adj_spec = pl.BlockSpec((BM, BM), _adj_map)
    feat_spec = pl.BlockSpec((BM, IN_DIM), _feat_map)
    w_spec = pl.BlockSpec((IN_DIM, D), lambda s: (0, 0))
    out_spec = pl.BlockSpec((BM, D), _out_map)

    user_h, biz_h = pl.pallas_call(
        _kernel,
        grid=(S1 + SU,),
        in_specs=[adj_spec, adj_spec, adj_spec, adj_spec,
                  feat_spec, feat_spec, w_spec, w_spec],
        out_specs=[out_spec, out_spec],
        out_shape=[
            jax.ShapeDtypeStruct((N, D), jnp.float32),
            jax.ShapeDtypeStruct((N, D), jnp.float32),
        ],
        scratch_shapes=[
            pltpu.VMEM((D, N), jnp.float32),
            pltpu.VMEM((D, N), jnp.float32),
            pltpu.VMEM((D, N), jnp.float32),
            pltpu.VMEM((D, N), jnp.float32),
            pltpu.VMEM((D, N), jnp.float32),
            pltpu.VMEM((D, N), jnp.float32),
        ],
        compiler_params=pltpu.CompilerParams(
            dimension_semantics=("arbitrary",),
        ),
    )(adj_bu, adj_uu, adj_ub, adj_bb, user_feat, biz_feat, W_user, W_biz)

    return (user_h, biz_h)


# speedup vs baseline: 1.4626x; 1.0552x over previous
import jax
import jax.numpy as jnp
from jax.experimental import pallas as pl
from jax.experimental.pallas import tpu as pltpu

N = 4096
D = 128
IN_DIM = 384
BM = 1024
NB = N // BM
S1 = NB * NB
SU = NB * (NB - 1) // 2 + 1
EPS = 1e-12


def _dot(a, x):
    return jax.lax.dot_general(
        a.astype(jnp.bfloat16), x.astype(jnp.bfloat16),
        (((1,), (0,)), ((), ())), preferred_element_type=jnp.float32,
    )


def _split(s):
    in_sweep = s < S1
    i1 = s // NB
    t = s - i1 * NB
    j1 = jax.lax.rem(i1 + 1 + t, NB)
    p = jnp.maximum(s - S1, 0)
    i2 = jnp.int32(0)
    for r in range(1, NB):
        i2 = i2 + (p >= (r * (2 * NB - 1 - r) // 2)).astype(jnp.int32)
    off = (i2 * (2 * NB - 1 - i2)) // 2
    j2r = i2 + 1 + (p - off)
    j2 = jnp.minimum(j2r, NB - 1)
    i = jnp.where(in_sweep, i1, i2)
    j = jnp.where(in_sweep, j1, j2)
    fin = jnp.logical_and(jnp.logical_not(in_sweep), j2r >= NB - 1)
    dummy = jnp.logical_and(jnp.logical_not(in_sweep), j2r > NB - 1)
    return in_sweep, i, j, t, fin, dummy


def _kernel(abu, auu, aub, abb, uf, bf, wu, wb, uh, bh,
            u0s, b0s, u1s, b1s, u2s, b2s):
    s = pl.program_id(0)
    in_sweep, i, j, t, fin, _ = _split(s)
    row = pl.ds(i * BM, BM)
    col = pl.ds(j * BM, BM)

    @pl.when(in_sweep)
    def _():
        @pl.when(i == 0)
        def _():
            u0s[col, :] = _dot(uf[...], wu[...])
            b0s[col, :] = _dot(bf[...], wb[...])

        @pl.when(t == 0)
        def _():
            z = jnp.zeros((BM, D), jnp.float32)
            u1s[row, :] = z
            b1s[row, :] = z
            u2s[row, :] = z
            b2s[row, :] = z

        u0j = u0s[col, :]
        b0j = b0s[col, :]

        @pl.when(j < i)
        def _():
            ustk = jnp.concatenate([u0j, u1s[col, :]], axis=1)
            bstk = jnp.concatenate([b0j, b1s[col, :]], axis=1)
            rbu = _dot(abu[...], bstk)
            ruu = _dot(auu[...], ustk)
            rub = _dot(aub[...], ustk)
            rbb = _dot(abb[...], bstk)
            u1s[row, :] += rbu[:, :D] + ruu[:, :D]
            b1s[row, :] += rub[:, :D] + rbb[:, :D]
            u2s[row, :] += rbu[:, D:] + ruu[:, D:]
            b2s[row, :] += rub[:, D:] + rbb[:, D:]

        @pl.when(j >= i)
        def _():
            u1s[row, :] += _dot(abu[...], b0j) + _dot(auu[...], u0j)
            b1s[row, :] += _dot(aub[...], u0j) + _dot(abb[...], b0j)

        @pl.when(t == NB - 1)
        def _():
            u1i = u1s[row, :]
            b1i = b1s[row, :]
            u2s[row, :] += _dot(abu[...], b1i) + _dot(auu[...], u1i)
            b2s[row, :] += _dot(aub[...], u1i) + _dot(abb[...], b1i)

    @pl.when(jnp.logical_not(in_sweep))
    def _():
        @pl.when(j > i)
        def _():
            u1j = u1s[col, :]
            b1j = b1s[col, :]
            u2s[row, :] += _dot(abu[...], b1j) + _dot(auu[...], u1j)
            b2s[row, :] += _dot(aub[...], u1j) + _dot(abb[...], b1j)

        @pl.when(fin)
        def _():
            emb_u = (u0s[row, :] + u1s[row, :] + u2s[row, :]) * (1.0 / 3.0)
            emb_b = (b0s[row, :] + b1s[row, :] + b2s[row, :]) * (1.0 / 3.0)
            nu = jnp.sqrt(jnp.sum(emb_u * emb_u, axis=1, keepdims=True))
            nb = jnp.sqrt(jnp.sum(emb_b * emb_b, axis=1, keepdims=True))
            uh[...] = emb_u / jnp.maximum(nu, EPS)
            bh[...] = emb_b / jnp.maximum(nb, EPS)


def _adj_map(s):
    _, i, j, _, _, dummy = _split(s)
    return (jnp.where(dummy, i - 1, i), j)


def _feat_map(s):
    return (jnp.where(s < NB, jax.lax.rem(s + 1, NB), 0), 0)


def _out_map(s):
    in_sweep, i, _, _, _, _ = _split(s)
    return (jnp.where(in_sweep, 0, i), 0)


def kernel(user_feat, biz_feat, adj_ub, adj_bu, adj_uu, adj_bb, W_user, W_biz):
    adj_spec = pl.BlockSpec((BM, BM), _adj_map)
    feat_spec = pl.BlockSpec((BM, IN_DIM), _feat_map)
    w_spec = pl.BlockSpec((IN_DIM, D), lambda s: (0, 0))
    out_spec = pl.BlockSpec((BM, D), _out_map)

    user_h, biz_h = pl.pallas_call(
        _kernel,
        grid=(S1 + SU,),
        in_specs=[adj_spec, adj_spec, adj_spec, adj_spec,
                  feat_spec, feat_spec, w_spec, w_spec],
        out_specs=[out_spec, out_spec],
        out_shape=[
            jax.ShapeDtypeStruct((N, D), jnp.float32),
            jax.ShapeDtypeStruct((N, D), jnp.float32),
        ],
        scratch_shapes=[
            pltpu.VMEM((N, D), jnp.float32),
            pltpu.VMEM((N, D), jnp.float32),
            pltpu.VMEM((N, D), jnp.float32),
            pltpu.VMEM((N, D), jnp.float32),
            pltpu.VMEM((N, D), jnp.float32),
            pltpu.VMEM((N, D), jnp.float32),
        ],
        compiler_params=pltpu.CompilerParams(
            dimension_semantics=("arbitrary",),
        ),
    )(adj_bu, adj_uu, adj_ub, adj_bb, user_feat, biz_feat, W_user, W_biz)

    return (user_h, biz_h)
